# Initial kernel scaffold; baseline (speedup 1.0000x reference)
import jax
import jax.numpy as jnp
from jax.experimental import pallas as pl


def kernel(input_ids, token_table, pos_table, gamma, beta):
    raise NotImplementedError("write your pallas kernel here")


# speedup vs baseline: 2.0227x; 2.0227x over previous
import functools

import jax
import jax.numpy as jnp
from jax import lax
from jax.experimental import pallas as pl
from jax.experimental.pallas import tpu as pltpu
from jax.experimental.pallas import tpu_sc as plsc

D = 128
NW = 32
CHUNK = 128
EPS = 1e-5
NGRP = D // 16


def _body(ids_hbm, table_hbm, pos_hbm, gamma_hbm, beta_hbm, out_hbm,
          idx_v, rows_v, pos_v, g_v, b_v, sem):
    n_tok = ids_hbm.shape[0]
    per_w = n_tok // NW
    n_chunks = per_w // CHUNK
    l_seq = pos_v.shape[0]

    wid = lax.axis_index("s") * 2 + lax.axis_index("c")
    base = wid * per_w

    pltpu.sync_copy(pos_hbm, pos_v)
    pltpu.sync_copy(gamma_hbm, g_v)
    pltpu.sync_copy(beta_hbm, b_v)
    g = [g_v[pl.ds(16 * j, 16)] for j in range(NGRP)]
    b = [b_v[pl.ds(16 * j, 16)] for j in range(NGRP)]
    inv_d = 1.0 / D
    perms = [jnp.arange(16, dtype=jnp.int32) ^ k for k in (1, 2, 4, 8)]

    def chunk_body(c, carry):
        rbase = base + c * CHUNK
        pltpu.sync_copy(ids_hbm.at[pl.ds(rbase, CHUNK)], idx_v)
        pltpu.async_copy(table_hbm.at[idx_v], rows_v, sem).wait()
        poff = (c * CHUNK) % l_seq

        def row_body(i, carry2):
            pi = poff + i
            pi = jnp.where(pi >= l_seq, pi - l_seq, pi)
            x = [rows_v[i, pl.ds(16 * j, 16)] + pos_v[pi, pl.ds(16 * j, 16)]
                 for j in range(NGRP)]
            s = ((x[0] + x[1]) + (x[2] + x[3])) + ((x[4] + x[5]) + (x[6] + x[7]))
            sq = [xj * xj for xj in x]
            q = ((sq[0] + sq[1]) + (sq[2] + sq[3])) + ((sq[4] + sq[5]) + (sq[6] + sq[7]))
            for p in perms:
                s = s + s.at[p].get(mode="promise_in_bounds", unique_indices=True)
                q = q + q.at[p].get(mode="promise_in_bounds", unique_indices=True)
            m = s * inv_d
            v = q * inv_d - m * m + EPS
            iv = lax.bitcast_convert_type(v, jnp.int32)
            magic = jnp.full((16,), 0x5F3759DF, dtype=jnp.int32)
            y = lax.bitcast_convert_type(
                magic - lax.shift_right_logical(iv, 1), jnp.float32)
            hv = 0.5 * v
            y = y * (1.5 - hv * y * y)
            y = y * (1.5 - hv * y * y)
            y = y * (1.5 - hv * y * y)
            for j in range(NGRP):
                rows_v[i, pl.ds(16 * j, 16)] = (x[j] - m) * y * g[j] + b[j]
            return carry2

        lax.fori_loop(0, CHUNK, row_body, 0)
        pltpu.sync_copy(rows_v, out_hbm.at[pl.ds(rbase, CHUNK)])
        return carry

    lax.fori_loop(0, n_chunks, chunk_body, 0)


@jax.jit
def kernel(input_ids, token_table, pos_table, gamma, beta):
    bsz, l_seq = input_ids.shape
    n_tok = bsz * l_seq
    ids_flat = input_ids.reshape(n_tok)
    pos = pos_table[:l_seq]

    mesh = plsc.VectorSubcoreMesh(core_axis_name="c", subcore_axis_name="s")
    run = pl.kernel(
        _body,
        mesh=mesh,
        out_type=jax.ShapeDtypeStruct((n_tok, D), jnp.float32),
        scratch_types=[
            pltpu.VMEM((CHUNK,), jnp.int32),
            pltpu.VMEM((CHUNK, D), jnp.float32),
            pltpu.VMEM((l_seq, D), jnp.float32),
            pltpu.VMEM((D,), jnp.float32),
            pltpu.VMEM((D,), jnp.float32),
            pltpu.SemaphoreType.DMA,
        ],
    )
    out = run(ids_flat, token_table, pos, gamma, beta)
    return out.reshape(bsz, l_seq, D)


# speedup vs baseline: 5.8115x; 2.8731x over previous
import functools

import jax
import jax.numpy as jnp
from jax import lax
from jax.experimental import pallas as pl
from jax.experimental.pallas import tpu as pltpu
from jax.experimental.pallas import tpu_sc as plsc

D = 128
NW = 32
CHUNK = 128
EPS = 1e-5
NGRP = D // 16


def _body(ids_hbm, table_hbm, pos_hbm, gamma_hbm, beta_hbm, out_hbm,
          idx_v, rows_v, pos_v, g_v, b_v, sem):
    n_tok = ids_hbm.shape[0]
    per_w = n_tok // NW
    n_chunks = per_w // CHUNK
    l_seq = pos_v.shape[0]

    wid = lax.axis_index("s") * 2 + lax.axis_index("c")
    base = wid * per_w

    pltpu.sync_copy(pos_hbm, pos_v)
    pltpu.sync_copy(gamma_hbm, g_v)
    pltpu.sync_copy(beta_hbm, b_v)
    g = [g_v[pl.ds(16 * j, 16)] for j in range(NGRP)]
    b = [b_v[pl.ds(16 * j, 16)] for j in range(NGRP)]
    inv_d = 1.0 / D
    perms = [jnp.arange(16, dtype=jnp.int32) ^ k for k in (1, 2, 4, 8)]

    def chunk_body(c, carry):
        rbase = base + c * CHUNK
        pltpu.sync_copy(ids_hbm.at[pl.ds(rbase, CHUNK)], idx_v)
        pltpu.async_copy(table_hbm.at[idx_v], rows_v, sem).wait()
        poff = (c * CHUNK) % l_seq

        @functools.partial(plsc.parallel_loop, 0, CHUNK, unroll=4)
        def row_body(i):
            pi = poff + i
            pi = jnp.where(pi >= l_seq, pi - l_seq, pi)
            x = [rows_v[i, pl.ds(16 * j, 16)] + pos_v[pi, pl.ds(16 * j, 16)]
                 for j in range(NGRP)]
            s = ((x[0] + x[1]) + (x[2] + x[3])) + ((x[4] + x[5]) + (x[6] + x[7]))
            sq = [xj * xj for xj in x]
            q = ((sq[0] + sq[1]) + (sq[2] + sq[3])) + ((sq[4] + sq[5]) + (sq[6] + sq[7]))
            for p in perms:
                s = s + s.at[p].get(mode="promise_in_bounds", unique_indices=True)
                q = q + q.at[p].get(mode="promise_in_bounds", unique_indices=True)
            m = s * inv_d
            v = q * inv_d - m * m + EPS
            iv = lax.bitcast_convert_type(v, jnp.int32)
            magic = jnp.full((16,), 0x5F3759DF, dtype=jnp.int32)
            y = lax.bitcast_convert_type(
                magic - lax.shift_right_logical(iv, 1), jnp.float32)
            hv = 0.5 * v
            y = y * (1.5 - hv * y * y)
            y = y * (1.5 - hv * y * y)
            y = y * (1.5 - hv * y * y)
            for j in range(NGRP):
                rows_v[i, pl.ds(16 * j, 16)] = (x[j] - m) * y * g[j] + b[j]

        pltpu.sync_copy(rows_v, out_hbm.at[pl.ds(rbase, CHUNK)])
        return carry

    lax.fori_loop(0, n_chunks, chunk_body, 0)


@jax.jit
def kernel(input_ids, token_table, pos_table, gamma, beta):
    bsz, l_seq = input_ids.shape
    n_tok = bsz * l_seq
    ids_flat = input_ids.reshape(n_tok)
    pos = pos_table[:l_seq]

    mesh = plsc.VectorSubcoreMesh(core_axis_name="c", subcore_axis_name="s")
    run = pl.kernel(
        _body,
        mesh=mesh,
        out_type=jax.ShapeDtypeStruct((n_tok, D), jnp.float32),
        scratch_types=[
            pltpu.VMEM((CHUNK,), jnp.int32),
            pltpu.VMEM((CHUNK, D), jnp.float32),
            pltpu.VMEM((l_seq, D), jnp.float32),
            pltpu.VMEM((D,), jnp.float32),
            pltpu.VMEM((D,), jnp.float32),
            pltpu.SemaphoreType.DMA,
        ],
    )
    out = run(ids_flat, token_table, pos, gamma, beta)
    return out.reshape(bsz, l_seq, D)
